# baseline (speedup 1.0000x reference)
import functools

import jax
import jax.numpy as jnp
from jax import lax
from jax.experimental import pallas as pl
from jax.experimental.pallas import tpu as pltpu
from jax.experimental.pallas import tpu_sc as plsc

_K = 64
_MASK = 10000.0
_LANE_CAP = 40
_CW = 16 * _LANE_CAP
_RG = 4


def _monotone_u32(x):
    b = lax.bitcast_convert_type(x, jnp.uint32)
    neg = b >= jnp.uint32(0x80000000)
    return jnp.where(neg, ~b, b | jnp.uint32(0x80000000))


def _inv_monotone_u32(u):
    pos = u >= jnp.uint32(0x80000000)
    b = jnp.where(pos, u ^ jnp.uint32(0x80000000), ~u)
    return lax.bitcast_convert_type(b, jnp.float32)


def _bisect_kth_u(u, k, iters=32):
    rows = u.shape[0]
    lo0 = jnp.zeros((rows, 1), jnp.uint32)
    hi0 = jnp.full((rows, 1), 0xFFFFFFFF, jnp.uint32)

    def body(_, carry):
        lo, hi = carry
        mid = lo + ((hi - lo) // 2) + ((hi - lo) & 1)
        cnt = jnp.sum((u >= mid).astype(jnp.int32), axis=1, keepdims=True)
        ok = cnt >= k
        return jnp.where(ok, mid, lo), jnp.where(ok, hi, mid - 1)

    t_u, _ = lax.fori_loop(0, iters, body, (lo0, hi0))
    return t_u


def _topk_expsum(s, m, t_u):
    u = _monotone_u32(s)
    t_f = _inv_monotone_u32(t_u)
    keep = u >= t_u
    cnt = jnp.sum(keep.astype(jnp.float32), axis=1, keepdims=True)
    sums = jnp.sum(jnp.where(keep, jnp.exp(s - m), 0.0), axis=1, keepdims=True)
    return sums - (cnt - float(_K)) * jnp.exp(t_f - m)


def _loss_terms(diag, m, sums):
    big = jnp.maximum(m, diag)
    lse = jnp.log(jnp.exp(diag - big) + sums * jnp.exp(m - big)) + big
    return lse - diag


def _stage1_body(q_ref, k_ref, s_out, t16_out, m_out, d_out, *, blk, bsz):
    g = pl.program_id(0)
    nblk = bsz // blk
    b = g % nblk

    q = q_ref[...]
    k = k_ref[...]
    s = lax.dot_general(q, k, (((1,), (1,)), ((), ())),
                        preferred_element_type=jnp.float32)

    rows = b * blk + lax.broadcasted_iota(jnp.int32, (blk, bsz), 0)
    cols = lax.broadcasted_iota(jnp.int32, (blk, bsz), 1)
    is_diag = rows == cols
    diag = jnp.sum(jnp.where(is_diag, s, 0.0), axis=1, keepdims=True)
    s = s - jnp.where(is_diag, _MASK, 0.0)
    s_out[...] = s

    f = s
    w = bsz
    for _ in range(4):
        w //= 2
        f = jnp.maximum(f[:, :w], f[:, w:])
    t_u = _bisect_kth_u(_monotone_u32(f), _K)
    t_f = _inv_monotone_u32(t_u)

    t16_out[...] = jnp.broadcast_to(t_f, (blk, 16))
    m_out[...] = jnp.max(f, axis=1, keepdims=True)
    d_out[...] = diag


def _stage2_body(s_hbm, t_hbm, cand_hbm, cnt_hbm, sbuf, cbuf, tbuf, cntbuf,
                 *, rows_per_w):
    nc = plsc.get_sparse_core_info().num_cores
    wid = lax.axis_index("s") * nc + lax.axis_index("c")
    base = wid * rows_per_w

    pltpu.sync_copy(t_hbm.at[pl.ds(base * 16, rows_per_w * 16)], tbuf)

    lanes = lax.iota(jnp.int32, 16)
    lane_base = lanes * _LANE_CAP
    neg_inf = jnp.full((16,), -jnp.inf, jnp.float32)
    ngroups = rows_per_w // _RG

    def group_body(g, acc):
        row0 = base + g * _RG
        pltpu.sync_copy(s_hbm.at[pl.ds(row0, _RG)], sbuf)

        for j in range(_RG * _CW // 16):
            cbuf[pl.ds(j * 16, 16)] = neg_inf

        for r in range(_RG):
            rl = g * _RG + r
            tb = tbuf[pl.ds(rl * 16, 16)]

            def chunk_body(i, cnt_vec, r=r, tb=tb):
                for u8 in range(8):
                    j = i * 8 + u8
                    x = sbuf[r, pl.ds(j * 16, 16)]
                    msk = x >= tb
                    okm = msk & (cnt_vec < _LANE_CAP)
                    idx = (r * _CW + lane_base
                           + jnp.minimum(cnt_vec, _LANE_CAP - 1))
                    plsc.store_scatter(cbuf, [idx], x, mask=okm)
                    cnt_vec = cnt_vec + msk.astype(jnp.int32)
                return cnt_vec

            cnt_vec = lax.fori_loop(0, 32, chunk_body,
                                    jnp.zeros((16,), jnp.int32))
            cmax = jnp.max(cnt_vec)
            acc = jnp.where(lanes == (g * _RG + r) % 16,
                            jnp.full((16,), cmax, jnp.int32), acc)

        @pl.when((g % 4) == 3)
        def _():
            cntbuf[pl.ds((g // 4) * 16, 16)] = acc

        pltpu.sync_copy(cbuf, cand_hbm.at[pl.ds(row0 * _CW, _RG * _CW)])
        return jnp.where((g % 4) == 3, jnp.zeros((16,), jnp.int32), acc)

    lax.fori_loop(0, ngroups, group_body, jnp.zeros((16,), jnp.int32))
    pltpu.sync_copy(cntbuf, cnt_hbm.at[pl.ds(base, rows_per_w)])


def _stage3_body(cand_ref, cnt_ref, m_ref, d_ref, q_ref, k_ref, out_ref,
                 *, blk, bsz):
    g = pl.program_id(0)
    nblk = bsz // blk
    b = g % nblk

    m = m_ref[...]
    diag = d_ref[...]
    overflow = jnp.max(cnt_ref[...]) > _LANE_CAP

    def cand_path(_):
        c = cand_ref[...]
        t_u = _bisect_kth_u(_monotone_u32(c), _K)
        return _loss_terms(diag, m, _topk_expsum(c, m, t_u))

    def full_path(_):
        q = q_ref[...]
        k = k_ref[...]
        s = lax.dot_general(q, k, (((1,), (1,)), ((), ())),
                            preferred_element_type=jnp.float32)
        rows = b * blk + lax.broadcasted_iota(jnp.int32, (blk, bsz), 0)
        cols = lax.broadcasted_iota(jnp.int32, (blk, bsz), 1)
        s = s - jnp.where(rows == cols, _MASK, 0.0)
        t_u = _bisect_kth_u(_monotone_u32(s), _K)
        return _loss_terms(diag, m, _topk_expsum(s, m, t_u))

    terms = lax.cond(overflow, full_path, cand_path, operand=None)
    part = jnp.sum(terms, keepdims=True) / float(bsz)

    @pl.when(g == 0)
    def _():
        out_ref[...] = jnp.zeros((1, 1), jnp.float32)

    out_ref[...] += part


def kernel(vis_feat, text_feat):
    bsz, dim = vis_feat.shape
    blk = 512
    nblk = bsz // blk
    nrows = 2 * bsz
    rows_per_w = nrows // 32

    q = jnp.concatenate([text_feat, vis_feat], axis=0)
    km = jnp.concatenate([vis_feat, text_feat], axis=0)

    s_hbm, t16, m, diag = pl.pallas_call(
        functools.partial(_stage1_body, blk=blk, bsz=bsz),
        grid=(2 * nblk,),
        in_specs=[
            pl.BlockSpec((blk, dim), lambda g: (g, 0)),
            pl.BlockSpec((bsz, dim), lambda g, nblk=nblk: (g // nblk, 0)),
        ],
        out_specs=[
            pl.BlockSpec((blk, bsz), lambda g: (g, 0)),
            pl.BlockSpec((blk, 16), lambda g: (g, 0)),
            pl.BlockSpec((blk, 1), lambda g: (g, 0)),
            pl.BlockSpec((blk, 1), lambda g: (g, 0)),
        ],
        out_shape=[
            jax.ShapeDtypeStruct((nrows, bsz), jnp.float32),
            jax.ShapeDtypeStruct((nrows, 16), jnp.float32),
            jax.ShapeDtypeStruct((nrows, 1), jnp.float32),
            jax.ShapeDtypeStruct((nrows, 1), jnp.float32),
        ],
    )(q, km)

    mesh = plsc.VectorSubcoreMesh(core_axis_name="c", subcore_axis_name="s")
    cand_flat, cnt = pl.kernel(
        functools.partial(_stage2_body, rows_per_w=rows_per_w),
        out_type=[
            jax.ShapeDtypeStruct((nrows * _CW,), jnp.float32),
            jax.ShapeDtypeStruct((nrows,), jnp.int32),
        ],
        mesh=mesh,
        compiler_params=pltpu.CompilerParams(needs_layout_passes=False),
        scratch_types=[
            pltpu.VMEM((_RG, bsz), jnp.float32),
            pltpu.VMEM((_RG * _CW,), jnp.float32),
            pltpu.VMEM((rows_per_w * 16,), jnp.float32),
            pltpu.VMEM((rows_per_w,), jnp.int32),
        ],
    )(s_hbm, t16.reshape(-1))

    out = pl.pallas_call(
        functools.partial(_stage3_body, blk=blk, bsz=bsz),
        grid=(2 * nblk,),
        in_specs=[
            pl.BlockSpec((blk, _CW), lambda g: (g, 0)),
            pl.BlockSpec((blk, 1), lambda g: (g, 0)),
            pl.BlockSpec((blk, 1), lambda g: (g, 0)),
            pl.BlockSpec((blk, 1), lambda g: (g, 0)),
            pl.BlockSpec((blk, dim), lambda g: (g, 0)),
            pl.BlockSpec((bsz, dim), lambda g, nblk=nblk: (g // nblk, 0)),
        ],
        out_specs=pl.BlockSpec((1, 1), lambda g: (0, 0)),
        out_shape=jax.ShapeDtypeStruct((1, 1), jnp.float32),
    )(cand_flat.reshape(nrows, _CW), cnt.reshape(nrows, 1), m, diag, q, km)
    return out[0, 0]


# speedup vs baseline: 1.2390x; 1.2390x over previous
import functools

import jax
import jax.numpy as jnp
from jax import lax
from jax.experimental import pallas as pl
from jax.experimental.pallas import tpu as pltpu
from jax.experimental.pallas import tpu_sc as plsc

_K = 64
_MASK = 10000.0
_LANE_CAP = 40
_CW = 16 * _LANE_CAP
_RG = 4


def _monotone_u32(x):
    b = lax.bitcast_convert_type(x, jnp.uint32)
    neg = b >= jnp.uint32(0x80000000)
    return jnp.where(neg, ~b, b | jnp.uint32(0x80000000))


def _inv_monotone_u32(u):
    pos = u >= jnp.uint32(0x80000000)
    b = jnp.where(pos, u ^ jnp.uint32(0x80000000), ~u)
    return lax.bitcast_convert_type(b, jnp.float32)


def _bisect_kth_u(u, k, iters=32):
    rows = u.shape[0]
    lo0 = jnp.zeros((rows, 1), jnp.uint32)
    hi0 = jnp.full((rows, 1), 0xFFFFFFFF, jnp.uint32)

    def body(_, carry):
        lo, hi = carry
        mid = lo + ((hi - lo) // 2) + ((hi - lo) & 1)
        cnt = jnp.sum((u >= mid).astype(jnp.int32), axis=1, keepdims=True)
        ok = cnt >= k
        return jnp.where(ok, mid, lo), jnp.where(ok, hi, mid - 1)

    t_u, _ = lax.fori_loop(0, iters, body, (lo0, hi0))
    return t_u


def _topk_expsum(s, m, t_u):
    u = _monotone_u32(s)
    t_f = _inv_monotone_u32(t_u)
    keep = u >= t_u
    cnt = jnp.sum(keep.astype(jnp.float32), axis=1, keepdims=True)
    sums = jnp.sum(jnp.where(keep, jnp.exp(s - m), 0.0), axis=1, keepdims=True)
    return sums - (cnt - float(_K)) * jnp.exp(t_f - m)


def _loss_terms(diag, m, sums):
    big = jnp.maximum(m, diag)
    lse = jnp.log(jnp.exp(diag - big) + sums * jnp.exp(m - big)) + big
    return lse - diag


def _stage1_body(q_ref, k_ref, s_out, t16_out, m_out, d_out, *, blk, bsz):
    g = pl.program_id(0)
    nblk = bsz // blk
    b = g % nblk

    q = q_ref[...]
    k = k_ref[...]
    s = lax.dot_general(q, k, (((1,), (1,)), ((), ())),
                        preferred_element_type=jnp.float32)

    rows = b * blk + lax.broadcasted_iota(jnp.int32, (blk, bsz), 0)
    cols = lax.broadcasted_iota(jnp.int32, (blk, bsz), 1)
    is_diag = rows == cols
    diag = jnp.sum(jnp.where(is_diag, s, 0.0), axis=1, keepdims=True)
    s = s - jnp.where(is_diag, _MASK, 0.0)
    s_out[...] = s

    f = s
    w = bsz
    for _ in range(4):
        w //= 2
        f = jnp.maximum(f[:, :w], f[:, w:])
    t_u = _bisect_kth_u(_monotone_u32(f), _K, iters=16)
    t_f = _inv_monotone_u32(t_u)

    t16_out[...] = jnp.broadcast_to(t_f, (blk, 16))
    m_out[...] = jnp.max(f, axis=1, keepdims=True)
    d_out[...] = diag


def _stage2_body(s_hbm, t_hbm, cand_hbm, cnt_hbm, sbuf0, sbuf1, cbuf0, cbuf1,
                 tbuf, cntbuf, ssem0, ssem1, csem0, csem1, *, rows_per_w):
    nc = plsc.get_sparse_core_info().num_cores
    wid = lax.axis_index("s") * nc + lax.axis_index("c")
    base = wid * rows_per_w

    pltpu.sync_copy(t_hbm.at[pl.ds(base * 16, rows_per_w * 16)], tbuf)

    lanes = lax.iota(jnp.int32, 16)
    neg_inf = jnp.full((16,), -jnp.inf, jnp.float32)
    ngroups = rows_per_w // _RG
    sbufs, cbufs = (sbuf0, sbuf1), (cbuf0, cbuf1)
    ssems, csems = (ssem0, ssem1), (csem0, csem1)

    def in_copy(g, b):
        return pltpu.make_async_copy(
            s_hbm.at[pl.ds(base + g * _RG, _RG)], sbufs[b], ssems[b])

    def out_copy(g, b):
        return pltpu.make_async_copy(
            cbufs[b], cand_hbm.at[pl.ds((base + g * _RG) * _CW, _RG * _CW)],
            csems[b])

    in_copy(0, 0).start()
    in_copy(1, 1).start()

    def pair_body(h, acc):
        for b in range(2):
            g = 2 * h + b
            sbuf, cbuf = sbufs[b], cbufs[b]

            @pl.when(h >= 1)
            def _(g=g, b=b):
                out_copy(g - 2, b).wait()

            in_copy(g, b).wait()

            for j in range(_RG * _CW // 16):
                cbuf[pl.ds(j * 16, 16)] = neg_inf

            for r in range(_RG):
                rl = g * _RG + r
                tb = tbuf[pl.ds(rl * 16, 16)]
                cnt0 = r * _CW + lanes * _LANE_CAP
                cap = cnt0 + (_LANE_CAP - 1)

                def chunk_body(i, cnt, r=r, tb=tb, cap=cap, sbuf=sbuf,
                               cbuf=cbuf):
                    for u8 in range(8):
                        j = i * 8 + u8
                        x = sbuf[r, pl.ds(j * 16, 16)]
                        msk = x >= tb
                        plsc.store_scatter(
                            cbuf, [jnp.minimum(cnt, cap)], x, mask=msk)
                        cnt = cnt + msk.astype(jnp.int32)
                    return cnt

                cnt_vec = lax.fori_loop(0, 32, chunk_body, cnt0)
                cmax = jnp.max(cnt_vec - cnt0)
                acc = jnp.where(lanes == (g * _RG + r) % 16,
                                jnp.full((16,), cmax, jnp.int32), acc)

            @pl.when((g % 4) == 3)
            def _(g=g):
                cntbuf[pl.ds((g // 4) * 16, 16)] = acc

            acc = jnp.where((g % 4) == 3, jnp.zeros((16,), jnp.int32), acc)

            out_copy(g, b).start()

            @pl.when(g + 2 < ngroups)
            def _(g=g, b=b):
                in_copy(g + 2, b).start()
        return acc

    lax.fori_loop(0, ngroups // 2, pair_body, jnp.zeros((16,), jnp.int32))
    out_copy(ngroups - 2, 0).wait()
    out_copy(ngroups - 1, 1).wait()
    pltpu.sync_copy(cntbuf, cnt_hbm.at[pl.ds(base, rows_per_w)])


def _stage3_body(cand_ref, cnt_ref, m_ref, d_ref, q_ref, k_ref, out_ref,
                 *, blk, bsz):
    g = pl.program_id(0)
    nblk = bsz // blk
    b = g % nblk

    m = m_ref[...]
    diag = d_ref[...]
    overflow = jnp.max(cnt_ref[...]) > _LANE_CAP

    def cand_path(_):
        c = cand_ref[...]
        t_u = _bisect_kth_u(_monotone_u32(c), _K)
        return _loss_terms(diag, m, _topk_expsum(c, m, t_u))

    def full_path(_):
        q = q_ref[...]
        k = k_ref[...]
        s = lax.dot_general(q, k, (((1,), (1,)), ((), ())),
                            preferred_element_type=jnp.float32)
        rows = b * blk + lax.broadcasted_iota(jnp.int32, (blk, bsz), 0)
        cols = lax.broadcasted_iota(jnp.int32, (blk, bsz), 1)
        s = s - jnp.where(rows == cols, _MASK, 0.0)
        t_u = _bisect_kth_u(_monotone_u32(s), _K)
        return _loss_terms(diag, m, _topk_expsum(s, m, t_u))

    terms = lax.cond(overflow, full_path, cand_path, operand=None)
    part = jnp.sum(terms, keepdims=True) / float(bsz)

    @pl.when(g == 0)
    def _():
        out_ref[...] = jnp.zeros((1, 1), jnp.float32)

    out_ref[...] += part


def kernel(vis_feat, text_feat):
    bsz, dim = vis_feat.shape
    blk = 512
    nblk = bsz // blk
    nrows = 2 * bsz
    rows_per_w = nrows // 32

    q = jnp.concatenate([text_feat, vis_feat], axis=0)
    km = jnp.concatenate([vis_feat, text_feat], axis=0)

    s_hbm, t16, m, diag = pl.pallas_call(
        functools.partial(_stage1_body, blk=blk, bsz=bsz),
        grid=(2 * nblk,),
        in_specs=[
            pl.BlockSpec((blk, dim), lambda g: (g, 0)),
            pl.BlockSpec((bsz, dim), lambda g, nblk=nblk: (g // nblk, 0)),
        ],
        out_specs=[
            pl.BlockSpec((blk, bsz), lambda g: (g, 0)),
            pl.BlockSpec((blk, 16), lambda g: (g, 0)),
            pl.BlockSpec((blk, 1), lambda g: (g, 0)),
            pl.BlockSpec((blk, 1), lambda g: (g, 0)),
        ],
        out_shape=[
            jax.ShapeDtypeStruct((nrows, bsz), jnp.float32),
            jax.ShapeDtypeStruct((nrows, 16), jnp.float32),
            jax.ShapeDtypeStruct((nrows, 1), jnp.float32),
            jax.ShapeDtypeStruct((nrows, 1), jnp.float32),
        ],
    )(q, km)

    mesh = plsc.VectorSubcoreMesh(core_axis_name="c", subcore_axis_name="s")
    cand_flat, cnt = pl.kernel(
        functools.partial(_stage2_body, rows_per_w=rows_per_w),
        out_type=[
            jax.ShapeDtypeStruct((nrows * _CW,), jnp.float32),
            jax.ShapeDtypeStruct((nrows,), jnp.int32),
        ],
        mesh=mesh,
        compiler_params=pltpu.CompilerParams(needs_layout_passes=False),
        scratch_types=[
            pltpu.VMEM((_RG, bsz), jnp.float32),
            pltpu.VMEM((_RG, bsz), jnp.float32),
            pltpu.VMEM((_RG * _CW,), jnp.float32),
            pltpu.VMEM((_RG * _CW,), jnp.float32),
            pltpu.VMEM((rows_per_w * 16,), jnp.float32),
            pltpu.VMEM((rows_per_w,), jnp.int32),
            pltpu.SemaphoreType.DMA,
            pltpu.SemaphoreType.DMA,
            pltpu.SemaphoreType.DMA,
            pltpu.SemaphoreType.DMA,
        ],
    )(s_hbm, t16.reshape(-1))

    out = pl.pallas_call(
        functools.partial(_stage3_body, blk=blk, bsz=bsz),
        grid=(2 * nblk,),
        in_specs=[
            pl.BlockSpec((blk, _CW), lambda g: (g, 0)),
            pl.BlockSpec((blk, 1), lambda g: (g, 0)),
            pl.BlockSpec((blk, 1), lambda g: (g, 0)),
            pl.BlockSpec((blk, 1), lambda g: (g, 0)),
            pl.BlockSpec((blk, dim), lambda g: (g, 0)),
            pl.BlockSpec((bsz, dim), lambda g, nblk=nblk: (g // nblk, 0)),
        ],
        out_specs=pl.BlockSpec((1, 1), lambda g: (0, 0)),
        out_shape=jax.ShapeDtypeStruct((1, 1), jnp.float32),
    )(cand_flat.reshape(nrows, _CW), cnt.reshape(nrows, 1), m, diag, q, km)
    return out[0, 0]


# speedup vs baseline: 1.3006x; 1.0498x over previous
import functools

import jax
import jax.numpy as jnp
from jax import lax
from jax.experimental import pallas as pl
from jax.experimental.pallas import tpu as pltpu
from jax.experimental.pallas import tpu_sc as plsc

_K = 64
_MASK = 10000.0
_LANE_CAP = 21
_CW = 16 * _LANE_CAP
_RG = 4


def _monotone_u32(x):
    b = lax.bitcast_convert_type(x, jnp.uint32)
    neg = b >= jnp.uint32(0x80000000)
    return jnp.where(neg, ~b, b | jnp.uint32(0x80000000))


def _inv_monotone_u32(u):
    pos = u >= jnp.uint32(0x80000000)
    b = jnp.where(pos, u ^ jnp.uint32(0x80000000), ~u)
    return lax.bitcast_convert_type(b, jnp.float32)


def _bisect_kth_u(u, k, iters=32):
    rows = u.shape[0]
    lo0 = jnp.zeros((rows, 1), jnp.uint32)
    hi0 = jnp.full((rows, 1), 0xFFFFFFFF, jnp.uint32)

    def body(_, carry):
        lo, hi = carry
        mid = lo + ((hi - lo) // 2) + ((hi - lo) & 1)
        cnt = jnp.sum((u >= mid).astype(jnp.int32), axis=1, keepdims=True)
        ok = cnt >= k
        return jnp.where(ok, mid, lo), jnp.where(ok, hi, mid - 1)

    t_u, _ = lax.fori_loop(0, iters, body, (lo0, hi0))
    return t_u


def _topk_expsum(s, m, t_u):
    u = _monotone_u32(s)
    t_f = _inv_monotone_u32(t_u)
    keep = u >= t_u
    cnt = jnp.sum(keep.astype(jnp.float32), axis=1, keepdims=True)
    sums = jnp.sum(jnp.where(keep, jnp.exp(s - m), 0.0), axis=1, keepdims=True)
    return sums - (cnt - float(_K)) * jnp.exp(t_f - m)


def _loss_terms(diag, m, sums):
    big = jnp.maximum(m, diag)
    lse = jnp.log(jnp.exp(diag - big) + sums * jnp.exp(m - big)) + big
    return lse - diag


def _stage1_body(q_ref, k_ref, s_out, t16_out, m_out, d_out, *, blk, bsz):
    g = pl.program_id(0)
    nblk = bsz // blk
    b = g % nblk

    q = q_ref[...]
    k = k_ref[...]
    s = lax.dot_general(q, k, (((1,), (1,)), ((), ())),
                        preferred_element_type=jnp.float32)

    rows = b * blk + lax.broadcasted_iota(jnp.int32, (blk, bsz), 0)
    cols = lax.broadcasted_iota(jnp.int32, (blk, bsz), 1)
    is_diag = rows == cols
    diag = jnp.sum(jnp.where(is_diag, s, 0.0), axis=1, keepdims=True)
    s = s - jnp.where(is_diag, _MASK, 0.0)
    s_out[...] = s

    f = s
    w = bsz
    for _ in range(4):
        w //= 2
        f = jnp.maximum(f[:, :w], f[:, w:])
    t_u = _bisect_kth_u(_monotone_u32(f), _K, iters=16)
    t_f = _inv_monotone_u32(t_u)

    t16_out[...] = jnp.broadcast_to(t_f, (blk, 16))
    m_out[...] = jnp.max(f, axis=1, keepdims=True)
    d_out[...] = diag


def _stage2_body(s_hbm, t_hbm, cand_hbm, cnt_hbm, sbuf0, sbuf1, cbuf0, cbuf1,
                 tbuf, cntbuf, ssem0, ssem1, csem0, csem1, *, rows_per_w):
    nc = plsc.get_sparse_core_info().num_cores
    wid = lax.axis_index("s") * nc + lax.axis_index("c")
    base = wid * rows_per_w

    pltpu.sync_copy(t_hbm.at[pl.ds(base * 16, rows_per_w * 16)], tbuf)

    lanes = lax.iota(jnp.int32, 16)
    neg_inf = jnp.full((16,), -jnp.inf, jnp.float32)
    ngroups = rows_per_w // _RG
    sbufs, cbufs = (sbuf0, sbuf1), (cbuf0, cbuf1)
    ssems, csems = (ssem0, ssem1), (csem0, csem1)

    def in_copy(g, b):
        return pltpu.make_async_copy(
            s_hbm.at[pl.ds(base + g * _RG, _RG)], sbufs[b], ssems[b])

    def out_copy(g, b):
        return pltpu.make_async_copy(
            cbufs[b], cand_hbm.at[pl.ds((base + g * _RG) * _CW, _RG * _CW)],
            csems[b])

    in_copy(0, 0).start()
    in_copy(1, 1).start()

    def pair_body(h, acc):
        for b in range(2):
            g = 2 * h + b
            sbuf, cbuf = sbufs[b], cbufs[b]

            @pl.when(h >= 1)
            def _(g=g, b=b):
                out_copy(g - 2, b).wait()

            in_copy(g, b).wait()

            for j in range(_RG * _CW // 16):
                cbuf[pl.ds(j * 16, 16)] = neg_inf

            for r in range(_RG):
                rl = g * _RG + r
                tb = tbuf[pl.ds(rl * 16, 16)]
                cnt0 = r * _CW + lanes * _LANE_CAP
                cap = cnt0 + (_LANE_CAP - 1)

                def chunk_body(i, cnt, r=r, tb=tb, cap=cap, sbuf=sbuf,
                               cbuf=cbuf):
                    for u8 in range(8):
                        j = i * 8 + u8
                        x = sbuf[r, pl.ds(j * 16, 16)]
                        msk = x >= tb
                        plsc.store_scatter(
                            cbuf, [jnp.minimum(cnt, cap)], x, mask=msk)
                        cnt = cnt + msk.astype(jnp.int32)
                    return cnt

                cnt_vec = lax.fori_loop(0, 32, chunk_body, cnt0)
                cmax = jnp.max(cnt_vec - cnt0)
                acc = jnp.where(lanes == (g * _RG + r) % 16,
                                jnp.full((16,), cmax, jnp.int32), acc)

            @pl.when((g % 4) == 3)
            def _(g=g):
                cntbuf[pl.ds((g // 4) * 16, 16)] = acc

            acc = jnp.where((g % 4) == 3, jnp.zeros((16,), jnp.int32), acc)

            out_copy(g, b).start()

            @pl.when(g + 2 < ngroups)
            def _(g=g, b=b):
                in_copy(g + 2, b).start()
        return acc

    lax.fori_loop(0, ngroups // 2, pair_body, jnp.zeros((16,), jnp.int32))
    out_copy(ngroups - 2, 0).wait()
    out_copy(ngroups - 1, 1).wait()
    pltpu.sync_copy(cntbuf, cnt_hbm.at[pl.ds(base, rows_per_w)])


def _stage3_body(cand_ref, cnt_ref, m_ref, d_ref, q_ref, k_ref, out_ref,
                 *, blk, bsz):
    g = pl.program_id(0)
    nblk = bsz // blk
    b = g % nblk

    m = m_ref[...]
    diag = d_ref[...]
    overflow = jnp.max(cnt_ref[...]) > _LANE_CAP

    def cand_path(_):
        c = cand_ref[...]
        t_u = _bisect_kth_u(_monotone_u32(c), _K)
        return _loss_terms(diag, m, _topk_expsum(c, m, t_u))

    def full_path(_):
        q = q_ref[...]
        k = k_ref[...]
        s = lax.dot_general(q, k, (((1,), (1,)), ((), ())),
                            preferred_element_type=jnp.float32)
        rows = b * blk + lax.broadcasted_iota(jnp.int32, (blk, bsz), 0)
        cols = lax.broadcasted_iota(jnp.int32, (blk, bsz), 1)
        s = s - jnp.where(rows == cols, _MASK, 0.0)
        t_u = _bisect_kth_u(_monotone_u32(s), _K)
        return _loss_terms(diag, m, _topk_expsum(s, m, t_u))

    terms = lax.cond(overflow, full_path, cand_path, operand=None)
    part = jnp.sum(terms, keepdims=True) / float(bsz)

    @pl.when(g == 0)
    def _():
        out_ref[...] = jnp.zeros((1, 1), jnp.float32)

    out_ref[...] += part


def kernel(vis_feat, text_feat):
    bsz, dim = vis_feat.shape
    blk = 512
    nblk = bsz // blk
    nrows = 2 * bsz
    rows_per_w = nrows // 32

    q = jnp.concatenate([text_feat, vis_feat], axis=0)
    km = jnp.concatenate([vis_feat, text_feat], axis=0)

    s_hbm, t16, m, diag = pl.pallas_call(
        functools.partial(_stage1_body, blk=blk, bsz=bsz),
        grid=(2 * nblk,),
        in_specs=[
            pl.BlockSpec((blk, dim), lambda g: (g, 0)),
            pl.BlockSpec((bsz, dim), lambda g, nblk=nblk: (g // nblk, 0)),
        ],
        out_specs=[
            pl.BlockSpec((blk, bsz), lambda g: (g, 0)),
            pl.BlockSpec((blk, 16), lambda g: (g, 0)),
            pl.BlockSpec((blk, 1), lambda g: (g, 0)),
            pl.BlockSpec((blk, 1), lambda g: (g, 0)),
        ],
        out_shape=[
            jax.ShapeDtypeStruct((nrows, bsz), jnp.float32),
            jax.ShapeDtypeStruct((nrows, 16), jnp.float32),
            jax.ShapeDtypeStruct((nrows, 1), jnp.float32),
            jax.ShapeDtypeStruct((nrows, 1), jnp.float32),
        ],
    )(q, km)

    mesh = plsc.VectorSubcoreMesh(core_axis_name="c", subcore_axis_name="s")
    cand_flat, cnt = pl.kernel(
        functools.partial(_stage2_body, rows_per_w=rows_per_w),
        out_type=[
            jax.ShapeDtypeStruct((nrows * _CW,), jnp.float32),
            jax.ShapeDtypeStruct((nrows,), jnp.int32),
        ],
        mesh=mesh,
        compiler_params=pltpu.CompilerParams(needs_layout_passes=False),
        scratch_types=[
            pltpu.VMEM((_RG, bsz), jnp.float32),
            pltpu.VMEM((_RG, bsz), jnp.float32),
            pltpu.VMEM((_RG * _CW,), jnp.float32),
            pltpu.VMEM((_RG * _CW,), jnp.float32),
            pltpu.VMEM((rows_per_w * 16,), jnp.float32),
            pltpu.VMEM((rows_per_w,), jnp.int32),
            pltpu.SemaphoreType.DMA,
            pltpu.SemaphoreType.DMA,
            pltpu.SemaphoreType.DMA,
            pltpu.SemaphoreType.DMA,
        ],
    )(s_hbm, t16.reshape(-1))

    out = pl.pallas_call(
        functools.partial(_stage3_body, blk=blk, bsz=bsz),
        grid=(2 * nblk,),
        in_specs=[
            pl.BlockSpec((blk, _CW), lambda g: (g, 0)),
            pl.BlockSpec((blk, 1), lambda g: (g, 0)),
            pl.BlockSpec((blk, 1), lambda g: (g, 0)),
            pl.BlockSpec((blk, 1), lambda g: (g, 0)),
            pl.BlockSpec((blk, dim), lambda g: (g, 0)),
            pl.BlockSpec((bsz, dim), lambda g, nblk=nblk: (g // nblk, 0)),
        ],
        out_specs=pl.BlockSpec((1, 1), lambda g: (0, 0)),
        out_shape=jax.ShapeDtypeStruct((1, 1), jnp.float32),
    )(cand_flat.reshape(nrows, _CW), cnt.reshape(nrows, 1), m, diag, q, km)
    return out[0, 0]


# speedup vs baseline: 1.3131x; 1.0096x over previous
import functools

import jax
import jax.numpy as jnp
from jax import lax
from jax.experimental import pallas as pl
from jax.experimental.pallas import tpu as pltpu
from jax.experimental.pallas import tpu_sc as plsc

_K = 64
_MASK = 10000.0
_LANE_CAP = 21
_CW = 16 * _LANE_CAP
_RG = 4


def _monotone_u32(x):
    b = lax.bitcast_convert_type(x, jnp.uint32)
    neg = b >= jnp.uint32(0x80000000)
    return jnp.where(neg, ~b, b | jnp.uint32(0x80000000))


def _inv_monotone_u32(u):
    pos = u >= jnp.uint32(0x80000000)
    b = jnp.where(pos, u ^ jnp.uint32(0x80000000), ~u)
    return lax.bitcast_convert_type(b, jnp.float32)


def _bisect_kth_u(u, k, iters=32):
    rows = u.shape[0]
    lo0 = jnp.zeros((rows, 1), jnp.uint32)
    hi0 = jnp.full((rows, 1), 0xFFFFFFFF, jnp.uint32)

    def body(_, carry):
        lo, hi = carry
        mid = lo + ((hi - lo) // 2) + ((hi - lo) & 1)
        cnt = jnp.sum((u >= mid).astype(jnp.int32), axis=1, keepdims=True)
        ok = cnt >= k
        return jnp.where(ok, mid, lo), jnp.where(ok, hi, mid - 1)

    t_u, _ = lax.fori_loop(0, iters, body, (lo0, hi0))
    return t_u


def _topk_expsum(s, m, t_u):
    u = _monotone_u32(s)
    t_f = _inv_monotone_u32(t_u)
    keep = u >= t_u
    cnt = jnp.sum(keep.astype(jnp.float32), axis=1, keepdims=True)
    sums = jnp.sum(jnp.where(keep, jnp.exp(s - m), 0.0), axis=1, keepdims=True)
    return sums - (cnt - float(_K)) * jnp.exp(t_f - m)


def _loss_terms(diag, m, sums):
    big = jnp.maximum(m, diag)
    lse = jnp.log(jnp.exp(diag - big) + sums * jnp.exp(m - big)) + big
    return lse - diag


def _stage1_body(q_ref, k_ref, s_out, t16_out, m_out, d_out, *, blk, bsz):
    g = pl.program_id(0)
    nblk = bsz // blk
    b = g % nblk

    q = q_ref[...]
    k = k_ref[...]
    s = lax.dot_general(q, k, (((1,), (1,)), ((), ())),
                        preferred_element_type=jnp.float32)

    rows = b * blk + lax.broadcasted_iota(jnp.int32, (blk, bsz), 0)
    cols = lax.broadcasted_iota(jnp.int32, (blk, bsz), 1)
    is_diag = rows == cols
    diag = jnp.sum(jnp.where(is_diag, s, 0.0), axis=1, keepdims=True)
    s = s - jnp.where(is_diag, _MASK, 0.0)
    s_out[...] = s

    f = s
    w = bsz
    for _ in range(4):
        w //= 2
        f = jnp.maximum(f[:, :w], f[:, w:])
    t_u = _bisect_kth_u(_monotone_u32(f), _K, iters=16)
    t_f = _inv_monotone_u32(t_u)

    t16_out[...] = jnp.broadcast_to(t_f, (blk, 16))
    m_out[...] = jnp.max(f, axis=1, keepdims=True)
    d_out[...] = diag


def _stage2_body(s_hbm, t_hbm, cand_hbm, cnt_hbm, sbuf0, sbuf1, cbuf0, cbuf1,
                 tbuf, cntbuf, ssem0, ssem1, csem0, csem1, *, rows_per_w):
    nc = plsc.get_sparse_core_info().num_cores
    wid = lax.axis_index("s") * nc + lax.axis_index("c")
    base = wid * rows_per_w

    pltpu.sync_copy(t_hbm.at[pl.ds(base * 16, rows_per_w * 16)], tbuf)

    lanes = lax.iota(jnp.int32, 16)
    neg_inf = jnp.full((16,), -jnp.inf, jnp.float32)
    ngroups = rows_per_w // _RG
    sbufs, cbufs = (sbuf0, sbuf1), (cbuf0, cbuf1)
    ssems, csems = (ssem0, ssem1), (csem0, csem1)

    def in_copy(g, b):
        return pltpu.make_async_copy(
            s_hbm.at[pl.ds(base + g * _RG, _RG)], sbufs[b], ssems[b])

    def out_copy(g, b):
        return pltpu.make_async_copy(
            cbufs[b], cand_hbm.at[pl.ds((base + g * _RG) * _CW, _RG * _CW)],
            csems[b])

    in_copy(0, 0).start()
    in_copy(1, 1).start()

    def pair_body(h, acc):
        for b in range(2):
            g = 2 * h + b
            sbuf, cbuf = sbufs[b], cbufs[b]

            @pl.when(h >= 1)
            def _(g=g, b=b):
                out_copy(g - 2, b).wait()

            in_copy(g, b).wait()

            for j in range(_RG * _CW // 16):
                cbuf[pl.ds(j * 16, 16)] = neg_inf

            for r in range(_RG):
                rl = g * _RG + r
                tb = tbuf[pl.ds(rl * 16, 16)]
                cnt0 = r * _CW + lanes * _LANE_CAP
                cap = cnt0 + (_LANE_CAP - 1)

                def chunk_body(i, cnt, r=r, tb=tb, cap=cap, sbuf=sbuf,
                               cbuf=cbuf):
                    for u8 in range(16):
                        j = i * 16 + u8
                        x = sbuf[r, pl.ds(j * 16, 16)]
                        msk = x >= tb
                        plsc.store_scatter(
                            cbuf, [jnp.minimum(cnt, cap)], x, mask=msk)
                        cnt = cnt + msk.astype(jnp.int32)
                    return cnt

                cnt_vec = lax.fori_loop(0, 16, chunk_body, cnt0)
                cmax = jnp.max(cnt_vec - cnt0)
                acc = jnp.where(lanes == (g * _RG + r) % 16,
                                jnp.full((16,), cmax, jnp.int32), acc)

            @pl.when((g % 4) == 3)
            def _(g=g):
                cntbuf[pl.ds((g // 4) * 16, 16)] = acc

            acc = jnp.where((g % 4) == 3, jnp.zeros((16,), jnp.int32), acc)

            out_copy(g, b).start()

            @pl.when(g + 2 < ngroups)
            def _(g=g, b=b):
                in_copy(g + 2, b).start()
        return acc

    lax.fori_loop(0, ngroups // 2, pair_body, jnp.zeros((16,), jnp.int32))
    out_copy(ngroups - 2, 0).wait()
    out_copy(ngroups - 1, 1).wait()
    pltpu.sync_copy(cntbuf, cnt_hbm.at[pl.ds(base, rows_per_w)])


def _stage3_body(cand_ref, m_ref, d_ref, out_ref, *, bsz):
    g = pl.program_id(0)
    m = m_ref[...]
    diag = d_ref[...]
    c = cand_ref[...]
    t_u = _bisect_kth_u(_monotone_u32(c), _K)
    terms = _loss_terms(diag, m, _topk_expsum(c, m, t_u))
    part = jnp.sum(terms, keepdims=True) / float(bsz)

    @pl.when(g == 0)
    def _():
        out_ref[...] = jnp.zeros((1, 1), jnp.float32)

    out_ref[...] += part


def _full_body(q_ref, k_ref, out_ref, *, blk, bsz):
    g = pl.program_id(0)
    b = g % (bsz // blk)
    q = q_ref[...]
    k = k_ref[...]
    s = lax.dot_general(q, k, (((1,), (1,)), ((), ())),
                        preferred_element_type=jnp.float32)
    rows = b * blk + lax.broadcasted_iota(jnp.int32, (blk, bsz), 0)
    cols = lax.broadcasted_iota(jnp.int32, (blk, bsz), 1)
    is_diag = rows == cols
    diag = jnp.sum(jnp.where(is_diag, s, 0.0), axis=1, keepdims=True)
    s = s - jnp.where(is_diag, _MASK, 0.0)
    m = jnp.max(s, axis=1, keepdims=True)
    t_u = _bisect_kth_u(_monotone_u32(s), _K)
    terms = _loss_terms(diag, m, _topk_expsum(s, m, t_u))
    part = jnp.sum(terms, keepdims=True) / float(bsz)

    @pl.when(g == 0)
    def _():
        out_ref[...] = jnp.zeros((1, 1), jnp.float32)

    out_ref[...] += part


def kernel(vis_feat, text_feat):
    bsz, dim = vis_feat.shape
    blk = 512
    nblk = bsz // blk
    nrows = 2 * bsz
    rows_per_w = nrows // 32

    q = jnp.concatenate([text_feat, vis_feat], axis=0)
    km = jnp.concatenate([vis_feat, text_feat], axis=0)

    s_hbm, t16, m, diag = pl.pallas_call(
        functools.partial(_stage1_body, blk=blk, bsz=bsz),
        grid=(2 * nblk,),
        in_specs=[
            pl.BlockSpec((blk, dim), lambda g: (g, 0)),
            pl.BlockSpec((bsz, dim), lambda g, nblk=nblk: (g // nblk, 0)),
        ],
        out_specs=[
            pl.BlockSpec((blk, bsz), lambda g: (g, 0)),
            pl.BlockSpec((blk, 16), lambda g: (g, 0)),
            pl.BlockSpec((blk, 1), lambda g: (g, 0)),
            pl.BlockSpec((blk, 1), lambda g: (g, 0)),
        ],
        out_shape=[
            jax.ShapeDtypeStruct((nrows, bsz), jnp.float32),
            jax.ShapeDtypeStruct((nrows, 16), jnp.float32),
            jax.ShapeDtypeStruct((nrows, 1), jnp.float32),
            jax.ShapeDtypeStruct((nrows, 1), jnp.float32),
        ],
    )(q, km)

    mesh = plsc.VectorSubcoreMesh(core_axis_name="c", subcore_axis_name="s")
    cand_flat, cnt = pl.kernel(
        functools.partial(_stage2_body, rows_per_w=rows_per_w),
        out_type=[
            jax.ShapeDtypeStruct((nrows * _CW,), jnp.float32),
            jax.ShapeDtypeStruct((nrows,), jnp.int32),
        ],
        mesh=mesh,
        compiler_params=pltpu.CompilerParams(needs_layout_passes=False),
        scratch_types=[
            pltpu.VMEM((_RG, bsz), jnp.float32),
            pltpu.VMEM((_RG, bsz), jnp.float32),
            pltpu.VMEM((_RG * _CW,), jnp.float32),
            pltpu.VMEM((_RG * _CW,), jnp.float32),
            pltpu.VMEM((rows_per_w * 16,), jnp.float32),
            pltpu.VMEM((rows_per_w,), jnp.int32),
            pltpu.SemaphoreType.DMA,
            pltpu.SemaphoreType.DMA,
            pltpu.SemaphoreType.DMA,
            pltpu.SemaphoreType.DMA,
        ],
    )(s_hbm, t16.reshape(-1))

    def cand_select(_):
        out = pl.pallas_call(
            functools.partial(_stage3_body, bsz=bsz),
            grid=(2 * nblk,),
            in_specs=[
                pl.BlockSpec((blk, _CW), lambda g: (g, 0)),
                pl.BlockSpec((blk, 1), lambda g: (g, 0)),
                pl.BlockSpec((blk, 1), lambda g: (g, 0)),
            ],
            out_specs=pl.BlockSpec((1, 1), lambda g: (0, 0)),
            out_shape=jax.ShapeDtypeStruct((1, 1), jnp.float32),
        )(cand_flat.reshape(nrows, _CW), m, diag)
        return out[0, 0]

    def full_select(_):
        out = pl.pallas_call(
            functools.partial(_full_body, blk=blk, bsz=bsz),
            grid=(2 * nblk,),
            in_specs=[
                pl.BlockSpec((blk, dim), lambda g: (g, 0)),
                pl.BlockSpec((bsz, dim), lambda g, nblk=nblk: (g // nblk, 0)),
            ],
            out_specs=pl.BlockSpec((1, 1), lambda g: (0, 0)),
            out_shape=jax.ShapeDtypeStruct((1, 1), jnp.float32),
        )(q, km)
        return out[0, 0]

    overflow = jnp.max(cnt) > _LANE_CAP
    return lax.cond(overflow, full_select, cand_select, operand=None)


# speedup vs baseline: 1.5572x; 1.1859x over previous
import functools

import jax
import jax.numpy as jnp
from jax import lax
from jax.experimental import pallas as pl
from jax.experimental.pallas import tpu as pltpu
from jax.experimental.pallas import tpu_sc as plsc

_K = 64
_MASK = 10000.0
_LANE_CAP = 21
_CW = 16 * _LANE_CAP
_RG = 4


def _monotone_u32(x):
    b = lax.bitcast_convert_type(x, jnp.uint32)
    neg = b >= jnp.uint32(0x80000000)
    return jnp.where(neg, ~b, b | jnp.uint32(0x80000000))


def _inv_monotone_u32(u):
    pos = u >= jnp.uint32(0x80000000)
    b = jnp.where(pos, u ^ jnp.uint32(0x80000000), ~u)
    return lax.bitcast_convert_type(b, jnp.float32)


def _bisect_kth_u(u, k, iters=32):
    rows = u.shape[0]
    lo0 = jnp.zeros((rows, 1), jnp.uint32)
    hi0 = jnp.full((rows, 1), 0xFFFFFFFF, jnp.uint32)

    def body(_, carry):
        lo, hi = carry
        mid = lo + ((hi - lo) // 2) + ((hi - lo) & 1)
        cnt = jnp.sum((u >= mid).astype(jnp.int32), axis=1, keepdims=True)
        ok = cnt >= k
        return jnp.where(ok, mid, lo), jnp.where(ok, hi, mid - 1)

    t_u, _ = lax.fori_loop(0, iters, body, (lo0, hi0))
    return t_u


def _topk_expsum(s, m, t_u):
    u = _monotone_u32(s)
    t_f = _inv_monotone_u32(t_u)
    keep = u >= t_u
    cnt = jnp.sum(keep.astype(jnp.float32), axis=1, keepdims=True)
    sums = jnp.sum(jnp.where(keep, jnp.exp(s - m), 0.0), axis=1, keepdims=True)
    return sums - (cnt - float(_K)) * jnp.exp(t_f - m)


def _loss_terms(diag, m, sums):
    big = jnp.maximum(m, diag)
    lse = jnp.log(jnp.exp(diag - big) + sums * jnp.exp(m - big)) + big
    return lse - diag


def _stage1_body(q_ref, k_ref, s_out, t16_out, m_out, d_out, *, blk, bsz):
    g = pl.program_id(0)
    nblk = bsz // blk
    b = g % nblk

    q = q_ref[...]
    k = k_ref[...]
    s = lax.dot_general(q, k, (((1,), (1,)), ((), ())),
                        preferred_element_type=jnp.float32)

    rows = b * blk + lax.broadcasted_iota(jnp.int32, (blk, bsz), 0)
    cols = lax.broadcasted_iota(jnp.int32, (blk, bsz), 1)
    is_diag = rows == cols
    diag = jnp.sum(jnp.where(is_diag, s, 0.0), axis=1, keepdims=True)
    s = s - jnp.where(is_diag, _MASK, 0.0)
    s_out[...] = s

    f = s
    w = bsz
    for _ in range(4):
        w //= 2
        f = jnp.maximum(f[:, :w], f[:, w:])
    t_u = _bisect_kth_u(_monotone_u32(f), _K, iters=16)
    t_f = _inv_monotone_u32(t_u)

    t16_out[...] = jnp.broadcast_to(t_f, (blk, 16))
    m_out[...] = jnp.max(f, axis=1, keepdims=True)
    d_out[...] = diag


def _stage2_body(s_hbm, t_hbm, cand_hbm, cnt_hbm, sbuf0, sbuf1, cbuf0, cbuf1,
                 tbuf, cntbuf, ssem0, ssem1, csem0, csem1, *, rows_per_w):
    nc = plsc.get_sparse_core_info().num_cores
    wid = lax.axis_index("s") * nc + lax.axis_index("c")
    base = wid * rows_per_w

    pltpu.sync_copy(t_hbm.at[pl.ds(base * 16, rows_per_w * 16)], tbuf)

    lanes = lax.iota(jnp.int32, 16)
    neg_inf = jnp.full((16,), -jnp.inf, jnp.float32)
    ngroups = rows_per_w // _RG
    sbufs, cbufs = (sbuf0, sbuf1), (cbuf0, cbuf1)
    ssems, csems = (ssem0, ssem1), (csem0, csem1)

    def in_copy(g, b):
        return pltpu.make_async_copy(
            s_hbm.at[pl.ds(base + g * _RG, _RG)], sbufs[b], ssems[b])

    def out_copy(g, b):
        return pltpu.make_async_copy(
            cbufs[b], cand_hbm.at[pl.ds((base + g * _RG) * _CW, _RG * _CW)],
            csems[b])

    in_copy(0, 0).start()
    in_copy(1, 1).start()

    def pair_body(h, acc):
        for b in range(2):
            g = 2 * h + b
            sbuf, cbuf = sbufs[b], cbufs[b]

            @pl.when(h >= 1)
            def _(g=g, b=b):
                out_copy(g - 2, b).wait()

            in_copy(g, b).wait()

            for j in range(_RG * _CW // 16):
                cbuf[pl.ds(j * 16, 16)] = neg_inf

            for r in range(_RG):
                rl = g * _RG + r
                tb = tbuf[pl.ds(rl * 16, 16)]
                cnt0 = r * _CW + lanes * _LANE_CAP
                cap = cnt0 + (_LANE_CAP - 1)

                def chunk_body(i, cnt, r=r, tb=tb, cap=cap, sbuf=sbuf,
                               cbuf=cbuf):
                    for u8 in range(16):
                        j = i * 16 + u8
                        x = sbuf[r, pl.ds(j * 16, 16)]
                        msk = x >= tb
                        plsc.store_scatter(
                            cbuf, [jnp.minimum(cnt, cap)], x, mask=msk)
                        cnt = cnt + msk.astype(jnp.int32)
                    return cnt

                cnt_vec = lax.fori_loop(0, 16, chunk_body, cnt0)
                cmax = jnp.max(cnt_vec - cnt0)
                acc = jnp.where(lanes == (g * _RG + r) % 16,
                                jnp.full((16,), cmax, jnp.int32), acc)

            @pl.when((g % 4) == 3)
            def _(g=g):
                cntbuf[pl.ds((g // 4) * 16, 16)] = acc

            acc = jnp.where((g % 4) == 3, jnp.zeros((16,), jnp.int32), acc)

            out_copy(g, b).start()

            @pl.when(g + 2 < ngroups)
            def _(g=g, b=b):
                in_copy(g + 2, b).start()
        return acc

    lax.fori_loop(0, ngroups // 2, pair_body, jnp.zeros((16,), jnp.int32))
    out_copy(ngroups - 2, 0).wait()
    out_copy(ngroups - 1, 1).wait()
    pltpu.sync_copy(cntbuf, cnt_hbm.at[pl.ds(base, rows_per_w)])


def _stage3_body(cand_ref, m_ref, d_ref, out_ref, *, blk, bsz):
    g = pl.program_id(0)
    ut = _monotone_u32(cand_ref[...]).T
    m = m_ref[...].reshape(1, blk)
    diag = d_ref[...].reshape(1, blk)

    lo0 = jnp.zeros((1, blk), jnp.uint32)
    hi0 = jnp.full((1, blk), 0xFFFFFFFF, jnp.uint32)

    def body(_, carry):
        lo, hi = carry
        mid = lo + ((hi - lo) // 2) + ((hi - lo) & 1)
        cnt = jnp.sum((ut >= mid).astype(jnp.int32), axis=0, keepdims=True)
        ok = cnt >= _K
        return jnp.where(ok, mid, lo), jnp.where(ok, hi, mid - 1)

    t_u, _ = lax.fori_loop(0, 32, body, (lo0, hi0))
    t_f = _inv_monotone_u32(t_u)

    ct = _inv_monotone_u32(ut)
    keep = ut >= t_u
    cnt = jnp.sum(keep.astype(jnp.float32), axis=0, keepdims=True)
    sums = jnp.sum(jnp.where(keep, jnp.exp(ct - m), 0.0), axis=0,
                   keepdims=True)
    sums = sums - (cnt - float(_K)) * jnp.exp(t_f - m)

    big = jnp.maximum(m, diag)
    lse = jnp.log(jnp.exp(diag - big) + sums * jnp.exp(m - big)) + big
    part = jnp.sum(lse - diag, keepdims=True) / float(bsz)

    @pl.when(g == 0)
    def _():
        out_ref[...] = jnp.zeros((1, 1), jnp.float32)

    out_ref[...] += part


def _full_body(q_ref, k_ref, out_ref, *, blk, bsz):
    g = pl.program_id(0)
    b = g % (bsz // blk)
    q = q_ref[...]
    k = k_ref[...]
    s = lax.dot_general(q, k, (((1,), (1,)), ((), ())),
                        preferred_element_type=jnp.float32)
    rows = b * blk + lax.broadcasted_iota(jnp.int32, (blk, bsz), 0)
    cols = lax.broadcasted_iota(jnp.int32, (blk, bsz), 1)
    is_diag = rows == cols
    diag = jnp.sum(jnp.where(is_diag, s, 0.0), axis=1, keepdims=True)
    s = s - jnp.where(is_diag, _MASK, 0.0)
    m = jnp.max(s, axis=1, keepdims=True)
    t_u = _bisect_kth_u(_monotone_u32(s), _K)
    terms = _loss_terms(diag, m, _topk_expsum(s, m, t_u))
    part = jnp.sum(terms, keepdims=True) / float(bsz)

    @pl.when(g == 0)
    def _():
        out_ref[...] = jnp.zeros((1, 1), jnp.float32)

    out_ref[...] += part


def kernel(vis_feat, text_feat):
    bsz, dim = vis_feat.shape
    blk = 512
    nblk = bsz // blk
    nrows = 2 * bsz
    rows_per_w = nrows // 32

    q = jnp.concatenate([text_feat, vis_feat], axis=0)
    km = jnp.concatenate([vis_feat, text_feat], axis=0)

    s_hbm, t16, m, diag = pl.pallas_call(
        functools.partial(_stage1_body, blk=blk, bsz=bsz),
        grid=(2 * nblk,),
        in_specs=[
            pl.BlockSpec((blk, dim), lambda g: (g, 0)),
            pl.BlockSpec((bsz, dim), lambda g, nblk=nblk: (g // nblk, 0)),
        ],
        out_specs=[
            pl.BlockSpec((blk, bsz), lambda g: (g, 0)),
            pl.BlockSpec((blk, 16), lambda g: (g, 0)),
            pl.BlockSpec((blk, 1), lambda g: (g, 0)),
            pl.BlockSpec((blk, 1), lambda g: (g, 0)),
        ],
        out_shape=[
            jax.ShapeDtypeStruct((nrows, bsz), jnp.float32),
            jax.ShapeDtypeStruct((nrows, 16), jnp.float32),
            jax.ShapeDtypeStruct((nrows, 1), jnp.float32),
            jax.ShapeDtypeStruct((nrows, 1), jnp.float32),
        ],
    )(q, km)

    mesh = plsc.VectorSubcoreMesh(core_axis_name="c", subcore_axis_name="s")
    cand_flat, cnt = pl.kernel(
        functools.partial(_stage2_body, rows_per_w=rows_per_w),
        out_type=[
            jax.ShapeDtypeStruct((nrows * _CW,), jnp.float32),
            jax.ShapeDtypeStruct((nrows,), jnp.int32),
        ],
        mesh=mesh,
        compiler_params=pltpu.CompilerParams(needs_layout_passes=False),
        scratch_types=[
            pltpu.VMEM((_RG, bsz), jnp.float32),
            pltpu.VMEM((_RG, bsz), jnp.float32),
            pltpu.VMEM((_RG * _CW,), jnp.float32),
            pltpu.VMEM((_RG * _CW,), jnp.float32),
            pltpu.VMEM((rows_per_w * 16,), jnp.float32),
            pltpu.VMEM((rows_per_w,), jnp.int32),
            pltpu.SemaphoreType.DMA,
            pltpu.SemaphoreType.DMA,
            pltpu.SemaphoreType.DMA,
            pltpu.SemaphoreType.DMA,
        ],
    )(s_hbm, t16.reshape(-1))

    def cand_select(_):
        out = pl.pallas_call(
            functools.partial(_stage3_body, blk=blk, bsz=bsz),
            grid=(2 * nblk,),
            in_specs=[
                pl.BlockSpec((blk, _CW), lambda g: (g, 0)),
                pl.BlockSpec((blk, 1), lambda g: (g, 0)),
                pl.BlockSpec((blk, 1), lambda g: (g, 0)),
            ],
            out_specs=pl.BlockSpec((1, 1), lambda g: (0, 0)),
            out_shape=jax.ShapeDtypeStruct((1, 1), jnp.float32),
        )(cand_flat.reshape(nrows, _CW), m, diag)
        return out[0, 0]

    def full_select(_):
        out = pl.pallas_call(
            functools.partial(_full_body, blk=blk, bsz=bsz),
            grid=(2 * nblk,),
            in_specs=[
                pl.BlockSpec((blk, dim), lambda g: (g, 0)),
                pl.BlockSpec((bsz, dim), lambda g, nblk=nblk: (g // nblk, 0)),
            ],
            out_specs=pl.BlockSpec((1, 1), lambda g: (0, 0)),
            out_shape=jax.ShapeDtypeStruct((1, 1), jnp.float32),
        )(q, km)
        return out[0, 0]

    overflow = jnp.max(cnt) > _LANE_CAP
    return lax.cond(overflow, full_select, cand_select, operand=None)


# speedup vs baseline: 1.6927x; 1.0870x over previous
import functools

import jax
import jax.numpy as jnp
from jax import lax
from jax.experimental import pallas as pl
from jax.experimental.pallas import tpu as pltpu
from jax.experimental.pallas import tpu_sc as plsc

_K = 64
_MASK = 10000.0
_LANE_CAP = 21
_CW = 16 * _LANE_CAP
_RG = 4


def _monotone_u32(x):
    b = lax.bitcast_convert_type(x, jnp.uint32)
    neg = b >= jnp.uint32(0x80000000)
    return jnp.where(neg, ~b, b | jnp.uint32(0x80000000))


def _inv_monotone_u32(u):
    pos = u >= jnp.uint32(0x80000000)
    b = jnp.where(pos, u ^ jnp.uint32(0x80000000), ~u)
    return lax.bitcast_convert_type(b, jnp.float32)


def _bisect_kth_u(u, k, iters=32):
    rows = u.shape[0]
    lo0 = jnp.zeros((rows, 1), jnp.uint32)
    hi0 = jnp.full((rows, 1), 0xFFFFFFFF, jnp.uint32)

    def body(_, carry):
        lo, hi = carry
        mid = lo + ((hi - lo) // 2) + ((hi - lo) & 1)
        cnt = jnp.sum((u >= mid).astype(jnp.int32), axis=1, keepdims=True)
        ok = cnt >= k
        return jnp.where(ok, mid, lo), jnp.where(ok, hi, mid - 1)

    t_u, _ = lax.fori_loop(0, iters, body, (lo0, hi0))
    return t_u


def _topk_expsum(s, m, t_u):
    u = _monotone_u32(s)
    t_f = _inv_monotone_u32(t_u)
    keep = u >= t_u
    cnt = jnp.sum(keep.astype(jnp.float32), axis=1, keepdims=True)
    sums = jnp.sum(jnp.where(keep, jnp.exp(s - m), 0.0), axis=1, keepdims=True)
    return sums - (cnt - float(_K)) * jnp.exp(t_f - m)


def _loss_terms(diag, m, sums):
    big = jnp.maximum(m, diag)
    lse = jnp.log(jnp.exp(diag - big) + sums * jnp.exp(m - big)) + big
    return lse - diag


def _stage1_body(q_ref, k_ref, s_out, t16_out, m_out, d_out, *, blk, bsz):
    g = pl.program_id(0)
    nblk = bsz // blk
    b = g % nblk

    q = q_ref[...]
    k = k_ref[...]
    s = lax.dot_general(q, k, (((1,), (1,)), ((), ())),
                        preferred_element_type=jnp.float32)

    rows = b * blk + lax.broadcasted_iota(jnp.int32, (blk, bsz), 0)
    cols = lax.broadcasted_iota(jnp.int32, (blk, bsz), 1)
    is_diag = rows == cols
    diag = jnp.sum(jnp.where(is_diag, s, 0.0), axis=1, keepdims=True)
    s = s - jnp.where(is_diag, _MASK, 0.0)
    s_out[...] = s

    f = s
    w = bsz
    for _ in range(4):
        w //= 2
        f = jnp.maximum(f[:, :w], f[:, w:])
    ft = _monotone_u32(f).T
    lo0 = jnp.zeros((1, blk), jnp.uint32)
    hi0 = jnp.full((1, blk), 0xFFFFFFFF, jnp.uint32)

    def bis(_, carry):
        lo, hi = carry
        mid = lo + ((hi - lo) // 2) + ((hi - lo) & 1)
        cnt = jnp.sum((ft >= mid).astype(jnp.int32), axis=0, keepdims=True)
        ok = cnt >= _K
        return jnp.where(ok, mid, lo), jnp.where(ok, hi, mid - 1)

    t_u, _ = lax.fori_loop(0, 16, bis, (lo0, hi0))
    t_f = _inv_monotone_u32(t_u)

    t16_out[...] = jnp.broadcast_to(t_f.T, (blk, 16))
    m_out[...] = jnp.max(f, axis=1, keepdims=True)
    d_out[...] = diag


def _stage2_body(s_hbm, t_hbm, cand_hbm, cnt_hbm, sbuf0, sbuf1, cbuf0, cbuf1,
                 tbuf, cntbuf, ssem0, ssem1, csem0, csem1, *, rows_per_w):
    nc = plsc.get_sparse_core_info().num_cores
    wid = lax.axis_index("s") * nc + lax.axis_index("c")
    base = wid * rows_per_w

    pltpu.sync_copy(t_hbm.at[pl.ds(base * 16, rows_per_w * 16)], tbuf)

    lanes = lax.iota(jnp.int32, 16)
    neg_inf = jnp.full((16,), -jnp.inf, jnp.float32)
    ngroups = rows_per_w // _RG
    sbufs, cbufs = (sbuf0, sbuf1), (cbuf0, cbuf1)
    ssems, csems = (ssem0, ssem1), (csem0, csem1)

    def in_copy(g, b):
        return pltpu.make_async_copy(
            s_hbm.at[pl.ds(base + g * _RG, _RG)], sbufs[b], ssems[b])

    def out_copy(g, b):
        return pltpu.make_async_copy(
            cbufs[b], cand_hbm.at[pl.ds((base + g * _RG) * _CW, _RG * _CW)],
            csems[b])

    in_copy(0, 0).start()
    in_copy(1, 1).start()

    def pair_body(h, acc):
        for b in range(2):
            g = 2 * h + b
            sbuf, cbuf = sbufs[b], cbufs[b]

            @pl.when(h >= 1)
            def _(g=g, b=b):
                out_copy(g - 2, b).wait()

            in_copy(g, b).wait()

            for j in range(_RG * _CW // 16):
                cbuf[pl.ds(j * 16, 16)] = neg_inf

            for r in range(_RG):
                rl = g * _RG + r
                tb = tbuf[pl.ds(rl * 16, 16)]
                cnt0 = r * _CW + lanes * _LANE_CAP
                cap = cnt0 + (_LANE_CAP - 1)

                def chunk_body(i, cnt, r=r, tb=tb, cap=cap, sbuf=sbuf,
                               cbuf=cbuf):
                    for u8 in range(16):
                        j = i * 16 + u8
                        x = sbuf[r, pl.ds(j * 16, 16)]
                        msk = x >= tb
                        plsc.store_scatter(
                            cbuf, [jnp.minimum(cnt, cap)], x, mask=msk)
                        cnt = cnt + msk.astype(jnp.int32)
                    return cnt

                cnt_vec = lax.fori_loop(0, 16, chunk_body, cnt0)
                cmax = jnp.max(cnt_vec - cnt0)
                acc = jnp.where(lanes == (g * _RG + r) % 16,
                                jnp.full((16,), cmax, jnp.int32), acc)

            @pl.when((g % 4) == 3)
            def _(g=g):
                cntbuf[pl.ds((g // 4) * 16, 16)] = acc

            acc = jnp.where((g % 4) == 3, jnp.zeros((16,), jnp.int32), acc)

            out_copy(g, b).start()

            @pl.when(g + 2 < ngroups)
            def _(g=g, b=b):
                in_copy(g + 2, b).start()
        return acc

    lax.fori_loop(0, ngroups // 2, pair_body, jnp.zeros((16,), jnp.int32))
    out_copy(ngroups - 2, 0).wait()
    out_copy(ngroups - 1, 1).wait()
    pltpu.sync_copy(cntbuf, cnt_hbm.at[pl.ds(base, rows_per_w)])


def _stage3_body(cand_ref, m_ref, d_ref, out_ref, *, blk, bsz):
    g = pl.program_id(0)
    ut = _monotone_u32(cand_ref[...]).T
    m = m_ref[...].reshape(1, blk)
    diag = d_ref[...].reshape(1, blk)

    lo0 = jnp.zeros((1, blk), jnp.uint32)
    hi0 = jnp.full((1, blk), 0xFFFFFFFF, jnp.uint32)

    def body(_, carry):
        lo, hi = carry
        mid = lo + ((hi - lo) // 2) + ((hi - lo) & 1)
        cnt = jnp.sum((ut >= mid).astype(jnp.int32), axis=0, keepdims=True)
        ok = cnt >= _K
        return jnp.where(ok, mid, lo), jnp.where(ok, hi, mid - 1)

    t_u, _ = lax.fori_loop(0, 32, body, (lo0, hi0))
    t_f = _inv_monotone_u32(t_u)

    ct = _inv_monotone_u32(ut)
    keep = ut >= t_u
    cnt = jnp.sum(keep.astype(jnp.float32), axis=0, keepdims=True)
    sums = jnp.sum(jnp.where(keep, jnp.exp(ct - m), 0.0), axis=0,
                   keepdims=True)
    sums = sums - (cnt - float(_K)) * jnp.exp(t_f - m)

    big = jnp.maximum(m, diag)
    lse = jnp.log(jnp.exp(diag - big) + sums * jnp.exp(m - big)) + big
    part = jnp.sum(lse - diag, keepdims=True) / float(bsz)

    @pl.when(g == 0)
    def _():
        out_ref[...] = jnp.zeros((1, 1), jnp.float32)

    out_ref[...] += part


def _full_body(q_ref, k_ref, out_ref, *, blk, bsz):
    g = pl.program_id(0)
    b = g % (bsz // blk)
    q = q_ref[...]
    k = k_ref[...]
    s = lax.dot_general(q, k, (((1,), (1,)), ((), ())),
                        preferred_element_type=jnp.float32)
    rows = b * blk + lax.broadcasted_iota(jnp.int32, (blk, bsz), 0)
    cols = lax.broadcasted_iota(jnp.int32, (blk, bsz), 1)
    is_diag = rows == cols
    diag = jnp.sum(jnp.where(is_diag, s, 0.0), axis=1, keepdims=True)
    s = s - jnp.where(is_diag, _MASK, 0.0)
    m = jnp.max(s, axis=1, keepdims=True)
    t_u = _bisect_kth_u(_monotone_u32(s), _K)
    terms = _loss_terms(diag, m, _topk_expsum(s, m, t_u))
    part = jnp.sum(terms, keepdims=True) / float(bsz)

    @pl.when(g == 0)
    def _():
        out_ref[...] = jnp.zeros((1, 1), jnp.float32)

    out_ref[...] += part


def kernel(vis_feat, text_feat):
    bsz, dim = vis_feat.shape
    blk = 512
    nblk = bsz // blk
    nrows = 2 * bsz
    rows_per_w = nrows // 32

    q = jnp.concatenate([text_feat, vis_feat], axis=0)
    km = jnp.concatenate([vis_feat, text_feat], axis=0)

    s_hbm, t16, m, diag = pl.pallas_call(
        functools.partial(_stage1_body, blk=blk, bsz=bsz),
        grid=(2 * nblk,),
        in_specs=[
            pl.BlockSpec((blk, dim), lambda g: (g, 0)),
            pl.BlockSpec((bsz, dim), lambda g, nblk=nblk: (g // nblk, 0)),
        ],
        out_specs=[
            pl.BlockSpec((blk, bsz), lambda g: (g, 0)),
            pl.BlockSpec((blk, 16), lambda g: (g, 0)),
            pl.BlockSpec((blk, 1), lambda g: (g, 0)),
            pl.BlockSpec((blk, 1), lambda g: (g, 0)),
        ],
        out_shape=[
            jax.ShapeDtypeStruct((nrows, bsz), jnp.float32),
            jax.ShapeDtypeStruct((nrows, 16), jnp.float32),
            jax.ShapeDtypeStruct((nrows, 1), jnp.float32),
            jax.ShapeDtypeStruct((nrows, 1), jnp.float32),
        ],
    )(q, km)

    mesh = plsc.VectorSubcoreMesh(core_axis_name="c", subcore_axis_name="s")
    cand_flat, cnt = pl.kernel(
        functools.partial(_stage2_body, rows_per_w=rows_per_w),
        out_type=[
            jax.ShapeDtypeStruct((nrows * _CW,), jnp.float32),
            jax.ShapeDtypeStruct((nrows,), jnp.int32),
        ],
        mesh=mesh,
        compiler_params=pltpu.CompilerParams(needs_layout_passes=False),
        scratch_types=[
            pltpu.VMEM((_RG, bsz), jnp.float32),
            pltpu.VMEM((_RG, bsz), jnp.float32),
            pltpu.VMEM((_RG * _CW,), jnp.float32),
            pltpu.VMEM((_RG * _CW,), jnp.float32),
            pltpu.VMEM((rows_per_w * 16,), jnp.float32),
            pltpu.VMEM((rows_per_w,), jnp.int32),
            pltpu.SemaphoreType.DMA,
            pltpu.SemaphoreType.DMA,
            pltpu.SemaphoreType.DMA,
            pltpu.SemaphoreType.DMA,
        ],
    )(s_hbm, t16.reshape(-1))

    def cand_select(_):
        out = pl.pallas_call(
            functools.partial(_stage3_body, blk=blk, bsz=bsz),
            grid=(2 * nblk,),
            in_specs=[
                pl.BlockSpec((blk, _CW), lambda g: (g, 0)),
                pl.BlockSpec((blk, 1), lambda g: (g, 0)),
                pl.BlockSpec((blk, 1), lambda g: (g, 0)),
            ],
            out_specs=pl.BlockSpec((1, 1), lambda g: (0, 0)),
            out_shape=jax.ShapeDtypeStruct((1, 1), jnp.float32),
        )(cand_flat.reshape(nrows, _CW), m, diag)
        return out[0, 0]

    def full_select(_):
        out = pl.pallas_call(
            functools.partial(_full_body, blk=blk, bsz=bsz),
            grid=(2 * nblk,),
            in_specs=[
                pl.BlockSpec((blk, dim), lambda g: (g, 0)),
                pl.BlockSpec((bsz, dim), lambda g, nblk=nblk: (g // nblk, 0)),
            ],
            out_specs=pl.BlockSpec((1, 1), lambda g: (0, 0)),
            out_shape=jax.ShapeDtypeStruct((1, 1), jnp.float32),
        )(q, km)
        return out[0, 0]

    overflow = jnp.max(cnt) > _LANE_CAP
    return lax.cond(overflow, full_select, cand_select, operand=None)


# speedup vs baseline: 2.3758x; 1.4035x over previous
import functools

import jax
import jax.numpy as jnp
from jax import lax
from jax.experimental import pallas as pl
from jax.experimental.pallas import tpu as pltpu
from jax.experimental.pallas import tpu_sc as plsc

_K = 64
_MASK = 10000.0
_LANE_CAP = 21
_CW = 16 * _LANE_CAP
_RG = 4


def _monotone_u32(x):
    b = lax.bitcast_convert_type(x, jnp.uint32)
    neg = b >= jnp.uint32(0x80000000)
    return jnp.where(neg, ~b, b | jnp.uint32(0x80000000))


def _inv_monotone_u32(u):
    pos = u >= jnp.uint32(0x80000000)
    b = jnp.where(pos, u ^ jnp.uint32(0x80000000), ~u)
    return lax.bitcast_convert_type(b, jnp.float32)


def _bisect_kth_u(u, k, iters=32):
    rows = u.shape[0]
    lo0 = jnp.zeros((rows, 1), jnp.uint32)
    hi0 = jnp.full((rows, 1), 0xFFFFFFFF, jnp.uint32)

    def body(_, carry):
        lo, hi = carry
        mid = lo + ((hi - lo) // 2) + ((hi - lo) & 1)
        cnt = jnp.sum((u >= mid).astype(jnp.int32), axis=1, keepdims=True)
        ok = cnt >= k
        return jnp.where(ok, mid, lo), jnp.where(ok, hi, mid - 1)

    t_u, _ = lax.fori_loop(0, iters, body, (lo0, hi0))
    return t_u


def _topk_expsum(s, m, t_u):
    u = _monotone_u32(s)
    t_f = _inv_monotone_u32(t_u)
    keep = u >= t_u
    cnt = jnp.sum(keep.astype(jnp.float32), axis=1, keepdims=True)
    sums = jnp.sum(jnp.where(keep, jnp.exp(s - m), 0.0), axis=1, keepdims=True)
    return sums - (cnt - float(_K)) * jnp.exp(t_f - m)


def _loss_terms(diag, m, sums):
    big = jnp.maximum(m, diag)
    lse = jnp.log(jnp.exp(diag - big) + sums * jnp.exp(m - big)) + big
    return lse - diag


def _stage1_body(q_ref, k_ref, s_out, t16_out, m_out, d_out, *, blk, bsz):
    g = pl.program_id(0)
    nblk = bsz // blk
    b = g % nblk

    q = q_ref[...]
    k = k_ref[...]
    s = lax.dot_general(q, k, (((1,), (1,)), ((), ())),
                        preferred_element_type=jnp.float32)

    rows = b * blk + lax.broadcasted_iota(jnp.int32, (blk, bsz), 0)
    cols = lax.broadcasted_iota(jnp.int32, (blk, bsz), 1)
    is_diag = rows == cols
    diag = jnp.sum(jnp.where(is_diag, s, 0.0), axis=1, keepdims=True)
    s = s - jnp.where(is_diag, _MASK, 0.0)
    s_out[...] = s

    f = s
    w = bsz
    for _ in range(4):
        w //= 2
        f = jnp.maximum(f[:, :w], f[:, w:])
    ft = _monotone_u32(f).T
    lo0 = jnp.zeros((1, blk), jnp.uint32)
    hi0 = jnp.full((1, blk), 0xFFFFFFFF, jnp.uint32)

    def bis(_, carry):
        lo, hi = carry
        mid = lo + ((hi - lo) // 2) + ((hi - lo) & 1)
        cnt = jnp.sum((ft >= mid).astype(jnp.int32), axis=0, keepdims=True)
        ok = cnt >= _K
        return jnp.where(ok, mid, lo), jnp.where(ok, hi, mid - 1)

    t_u, _ = lax.fori_loop(0, 16, bis, (lo0, hi0))
    t_f = _inv_monotone_u32(t_u)

    t16_out[...] = jnp.broadcast_to(t_f.T, (blk, 16))
    m_out[...] = jnp.max(f, axis=1, keepdims=True)
    d_out[...] = diag


def _stage2_body(s_hbm, t_hbm, cand_hbm, cnt_hbm, sbuf0, sbuf1, cbuf0, cbuf1,
                 tbuf, cntbuf, ssem0, ssem1, csem0, csem1, *, rows_per_w):
    nc = plsc.get_sparse_core_info().num_cores
    wid = lax.axis_index("s") * nc + lax.axis_index("c")
    base = wid * rows_per_w

    pltpu.sync_copy(t_hbm.at[pl.ds(base * 16, rows_per_w * 16)], tbuf)

    lanes = lax.iota(jnp.int32, 16)
    neg_inf = jnp.full((16,), -jnp.inf, jnp.float32)
    ngroups = rows_per_w // _RG
    sbufs, cbufs = (sbuf0, sbuf1), (cbuf0, cbuf1)
    ssems, csems = (ssem0, ssem1), (csem0, csem1)

    def in_copy(g, b):
        return pltpu.make_async_copy(
            s_hbm.at[pl.ds(base + g * _RG, _RG)], sbufs[b], ssems[b])

    def out_copy(g, b):
        return pltpu.make_async_copy(
            cbufs[b], cand_hbm.at[pl.ds((base + g * _RG) * _CW, _RG * _CW)],
            csems[b])

    in_copy(0, 0).start()
    in_copy(1, 1).start()

    def pair_body(h, acc):
        for b in range(2):
            g = 2 * h + b
            sbuf, cbuf = sbufs[b], cbufs[b]

            @pl.when(h >= 1)
            def _(g=g, b=b):
                out_copy(g - 2, b).wait()

            in_copy(g, b).wait()

            for j in range(_RG * _CW // 16):
                cbuf[pl.ds(j * 16, 16)] = neg_inf

            for r in range(_RG):
                rl = g * _RG + r
                tb = tbuf[pl.ds(rl * 16, 16)]
                cnt0 = r * _CW + lanes * _LANE_CAP
                cap = cnt0 + (_LANE_CAP - 1)

                @plsc.parallel_loop(0, 256, unroll=16, carry=cnt0)
                def cnt_vec(j, cnt, r=r, tb=tb, cap=cap, sbuf=sbuf,
                            cbuf=cbuf):
                    x = sbuf[r, pl.ds(j * 16, 16)]
                    msk = x >= tb
                    plsc.store_scatter(
                        cbuf, [jnp.minimum(cnt, cap)], x, mask=msk)
                    return cnt + msk.astype(jnp.int32)
                cmax = jnp.max(cnt_vec - cnt0)
                acc = jnp.where(lanes == (g * _RG + r) % 16,
                                jnp.full((16,), cmax, jnp.int32), acc)

            @pl.when((g % 4) == 3)
            def _(g=g):
                cntbuf[pl.ds((g // 4) * 16, 16)] = acc

            acc = jnp.where((g % 4) == 3, jnp.zeros((16,), jnp.int32), acc)

            out_copy(g, b).start()

            @pl.when(g + 2 < ngroups)
            def _(g=g, b=b):
                in_copy(g + 2, b).start()
        return acc

    lax.fori_loop(0, ngroups // 2, pair_body, jnp.zeros((16,), jnp.int32))
    out_copy(ngroups - 2, 0).wait()
    out_copy(ngroups - 1, 1).wait()
    pltpu.sync_copy(cntbuf, cnt_hbm.at[pl.ds(base, rows_per_w)])


def _stage3_body(cand_ref, m_ref, d_ref, out_ref, *, blk, bsz):
    g = pl.program_id(0)
    ut = _monotone_u32(cand_ref[...]).T
    m = m_ref[...].reshape(1, blk)
    diag = d_ref[...].reshape(1, blk)

    lo0 = jnp.zeros((1, blk), jnp.uint32)
    hi0 = jnp.full((1, blk), 0xFFFFFFFF, jnp.uint32)

    def body(_, carry):
        lo, hi = carry
        mid = lo + ((hi - lo) // 2) + ((hi - lo) & 1)
        cnt = jnp.sum((ut >= mid).astype(jnp.int32), axis=0, keepdims=True)
        ok = cnt >= _K
        return jnp.where(ok, mid, lo), jnp.where(ok, hi, mid - 1)

    t_u, _ = lax.fori_loop(0, 32, body, (lo0, hi0))
    t_f = _inv_monotone_u32(t_u)

    ct = _inv_monotone_u32(ut)
    keep = ut >= t_u
    cnt = jnp.sum(keep.astype(jnp.float32), axis=0, keepdims=True)
    sums = jnp.sum(jnp.where(keep, jnp.exp(ct - m), 0.0), axis=0,
                   keepdims=True)
    sums = sums - (cnt - float(_K)) * jnp.exp(t_f - m)

    big = jnp.maximum(m, diag)
    lse = jnp.log(jnp.exp(diag - big) + sums * jnp.exp(m - big)) + big
    part = jnp.sum(lse - diag, keepdims=True) / float(bsz)

    @pl.when(g == 0)
    def _():
        out_ref[...] = jnp.zeros((1, 1), jnp.float32)

    out_ref[...] += part


def _full_body(q_ref, k_ref, out_ref, *, blk, bsz):
    g = pl.program_id(0)
    b = g % (bsz // blk)
    q = q_ref[...]
    k = k_ref[...]
    s = lax.dot_general(q, k, (((1,), (1,)), ((), ())),
                        preferred_element_type=jnp.float32)
    rows = b * blk + lax.broadcasted_iota(jnp.int32, (blk, bsz), 0)
    cols = lax.broadcasted_iota(jnp.int32, (blk, bsz), 1)
    is_diag = rows == cols
    diag = jnp.sum(jnp.where(is_diag, s, 0.0), axis=1, keepdims=True)
    s = s - jnp.where(is_diag, _MASK, 0.0)
    m = jnp.max(s, axis=1, keepdims=True)
    t_u = _bisect_kth_u(_monotone_u32(s), _K)
    terms = _loss_terms(diag, m, _topk_expsum(s, m, t_u))
    part = jnp.sum(terms, keepdims=True) / float(bsz)

    @pl.when(g == 0)
    def _():
        out_ref[...] = jnp.zeros((1, 1), jnp.float32)

    out_ref[...] += part


def kernel(vis_feat, text_feat):
    bsz, dim = vis_feat.shape
    blk = 512
    nblk = bsz // blk
    nrows = 2 * bsz
    rows_per_w = nrows // 32

    q = jnp.concatenate([text_feat, vis_feat], axis=0)
    km = jnp.concatenate([vis_feat, text_feat], axis=0)

    s_hbm, t16, m, diag = pl.pallas_call(
        functools.partial(_stage1_body, blk=blk, bsz=bsz),
        grid=(2 * nblk,),
        in_specs=[
            pl.BlockSpec((blk, dim), lambda g: (g, 0)),
            pl.BlockSpec((bsz, dim), lambda g, nblk=nblk: (g // nblk, 0)),
        ],
        out_specs=[
            pl.BlockSpec((blk, bsz), lambda g: (g, 0)),
            pl.BlockSpec((blk, 16), lambda g: (g, 0)),
            pl.BlockSpec((blk, 1), lambda g: (g, 0)),
            pl.BlockSpec((blk, 1), lambda g: (g, 0)),
        ],
        out_shape=[
            jax.ShapeDtypeStruct((nrows, bsz), jnp.float32),
            jax.ShapeDtypeStruct((nrows, 16), jnp.float32),
            jax.ShapeDtypeStruct((nrows, 1), jnp.float32),
            jax.ShapeDtypeStruct((nrows, 1), jnp.float32),
        ],
    )(q, km)

    mesh = plsc.VectorSubcoreMesh(core_axis_name="c", subcore_axis_name="s")
    cand_flat, cnt = pl.kernel(
        functools.partial(_stage2_body, rows_per_w=rows_per_w),
        out_type=[
            jax.ShapeDtypeStruct((nrows * _CW,), jnp.float32),
            jax.ShapeDtypeStruct((nrows,), jnp.int32),
        ],
        mesh=mesh,
        compiler_params=pltpu.CompilerParams(needs_layout_passes=False),
        scratch_types=[
            pltpu.VMEM((_RG, bsz), jnp.float32),
            pltpu.VMEM((_RG, bsz), jnp.float32),
            pltpu.VMEM((_RG * _CW,), jnp.float32),
            pltpu.VMEM((_RG * _CW,), jnp.float32),
            pltpu.VMEM((rows_per_w * 16,), jnp.float32),
            pltpu.VMEM((rows_per_w,), jnp.int32),
            pltpu.SemaphoreType.DMA,
            pltpu.SemaphoreType.DMA,
            pltpu.SemaphoreType.DMA,
            pltpu.SemaphoreType.DMA,
        ],
    )(s_hbm, t16.reshape(-1))

    def cand_select(_):
        out = pl.pallas_call(
            functools.partial(_stage3_body, blk=blk, bsz=bsz),
            grid=(2 * nblk,),
            in_specs=[
                pl.BlockSpec((blk, _CW), lambda g: (g, 0)),
                pl.BlockSpec((blk, 1), lambda g: (g, 0)),
                pl.BlockSpec((blk, 1), lambda g: (g, 0)),
            ],
            out_specs=pl.BlockSpec((1, 1), lambda g: (0, 0)),
            out_shape=jax.ShapeDtypeStruct((1, 1), jnp.float32),
        )(cand_flat.reshape(nrows, _CW), m, diag)
        return out[0, 0]

    def full_select(_):
        out = pl.pallas_call(
            functools.partial(_full_body, blk=blk, bsz=bsz),
            grid=(2 * nblk,),
            in_specs=[
                pl.BlockSpec((blk, dim), lambda g: (g, 0)),
                pl.BlockSpec((bsz, dim), lambda g, nblk=nblk: (g // nblk, 0)),
            ],
            out_specs=pl.BlockSpec((1, 1), lambda g: (0, 0)),
            out_shape=jax.ShapeDtypeStruct((1, 1), jnp.float32),
        )(q, km)
        return out[0, 0]

    overflow = jnp.max(cnt) > _LANE_CAP
    return lax.cond(overflow, full_select, cand_select, operand=None)
